# baseline (speedup 1.0000x reference)
import functools

import jax
import jax.numpy as jnp
from jax import lax
from jax.experimental import pallas as pl
from jax.experimental.pallas import tpu as pltpu
from jax.experimental.pallas import tpu_sc as plsc

BATCH = 16384
D = 64
NUM_CORES = 2
NUM_SUBCORES = 16
NW = NUM_CORES * NUM_SUBCORES
BPW = BATCH // NW
CHUNK = 128
NCH = BPW // CHUNK
GROUPS = BPW // 16


def _sc_scores(user_idx, pos_idx, neg_idx, user_emb, item_emb):
    mesh = plsc.VectorSubcoreMesh(core_axis_name="c", subcore_axis_name="s")

    @functools.partial(
        pl.kernel,
        mesh=mesh,
        out_type=jax.ShapeDtypeStruct((BATCH * 16,), jnp.float32),
        compiler_params=pltpu.CompilerParams(use_tc_tiling_on_sc=False),
        scratch_types=[
            pltpu.VMEM((BPW,), jnp.int32),
            pltpu.VMEM((BPW,), jnp.int32),
            pltpu.VMEM((BPW,), jnp.int32),
            pltpu.VMEM((BPW, D), jnp.float32),
            pltpu.VMEM((BPW, D), jnp.float32),
            pltpu.VMEM((BPW, D), jnp.float32),
            pltpu.VMEM((BPW * 16,), jnp.float32),
            pltpu.SemaphoreType.DMA,
        ],
    )
    def k(uidx_h, pidx_h, nidx_h, uemb_h, iemb_h, out_h,
          uidx_v, pidx_v, nidx_v, urows, prows, nrows, scores, sem):
        wid = lax.axis_index("s") * NUM_CORES + lax.axis_index("c")
        base = wid * BPW

        pltpu.sync_copy(uidx_h.at[pl.ds(base, BPW)], uidx_v)
        pltpu.sync_copy(pidx_h.at[pl.ds(base, BPW)], pidx_v)
        pltpu.sync_copy(nidx_h.at[pl.ds(base, BPW)], nidx_v)

        handles = []
        for j in range(NCH):
            sl = pl.ds(j * CHUNK, CHUNK)
            handles.append(pltpu.async_copy(uemb_h.at[uidx_v.at[sl]], urows.at[sl], sem))
            handles.append(pltpu.async_copy(iemb_h.at[pidx_v.at[sl]], prows.at[sl], sem))
            handles.append(pltpu.async_copy(iemb_h.at[nidx_v.at[sl]], nrows.at[sl], sem))
        for h in handles:
            h.wait()

        def row_body(b, carry):
            s = jnp.zeros((16,), jnp.float32)
            for k2 in range(D // 16):
                sl = pl.ds(k2 * 16, 16)
                s = s + urows[b, sl] * (prows[b, sl] - nrows[b, sl])
            scores[pl.ds(b * 16, 16)] = s
            return carry

        lax.fori_loop(0, BPW, row_body, 0)

        pltpu.sync_copy(scores, out_h.at[pl.ds(base * 16, BPW * 16)])

    return k(user_idx, pos_idx, neg_idx, user_emb, item_emb)


def _tc_loss(partials_2d):
    def body(x_ref, o_ref):
        r = lax.broadcasted_iota(jnp.int32, (128, 128), 0) // 16
        c = lax.broadcasted_iota(jnp.int32, (128, 128), 1) // 16
        m = (r == c).astype(jnp.float32)
        y = jnp.dot(x_ref[...], m, preferred_element_type=jnp.float32,
                    precision=jax.lax.Precision.HIGHEST)
        t = -y
        sp = jnp.maximum(t, 0.0) + jnp.log(1.0 + jnp.exp(-jnp.abs(t)))
        o_ref[0, 0] = jnp.sum(sp) / (16.0 * BATCH)

    out = pl.pallas_call(
        body,
        out_shape=jax.ShapeDtypeStruct((1, 1), jnp.float32),
        out_specs=pl.BlockSpec(memory_space=pltpu.SMEM),
    )(partials_2d)
    return out[0, 0]


def kernel(user_idx, pos_idx, neg_idx, user_emb, item_emb):
    partials = _sc_scores(user_idx, pos_idx, neg_idx, user_emb, item_emb)
    return _tc_loss(partials.reshape(2048, 128))
